# baseline (speedup 1.0000x reference)
import functools

import jax
import jax.numpy as jnp
from jax import lax
from jax.experimental import pallas as pl
from jax.experimental.pallas import tpu as pltpu
from jax.experimental.pallas import tpu_sc as plsc

_N = 10000
_D = 128
_E = 320000
_NC = 2
_NS = 16
_NW = _NC * _NS
_CH = 64
_NCH = 160
_EPT = _NCH * _CH
_E_PAD = _EPT * _NW
_N_PAD = 10112
_RPS = _N_PAD // _NS


def _h_body(nf, cnd, fw, fb, cv, cg, cb, out):
    hh = jnp.dot(nf[...], fw[...], preferred_element_type=jnp.float32) + fb[...]
    mu = jnp.mean(hh, axis=-1, keepdims=True)
    var = jnp.mean((hh - mu) * (hh - mu), axis=-1, keepdims=True)
    hn = (hh - mu) * lax.rsqrt(var + 1e-5)
    v = cv[...]
    norm = jnp.sqrt(jnp.sum(v * v, axis=0, keepdims=True) + 1e-12)
    w = v * (cg[...] / norm)
    gb = jnp.dot(cnd[...], w, preferred_element_type=jnp.float32) + cb[...]
    gamma = gb[:, :_D] + 1.0
    beta = gb[:, _D:]
    out[...] = jnp.maximum(gamma * hn + beta, 0.0)


def _compute_h(node_feats, cond, film_w, film_b, cond_v, cond_g, cond_b):
    bn = 1000
    grid = (_N // bn,)
    full = lambda shape: pl.BlockSpec(shape, lambda i: (0, 0))
    return pl.pallas_call(
        _h_body,
        grid=grid,
        in_specs=[
            pl.BlockSpec((bn, _D), lambda i: (i, 0)),
            pl.BlockSpec((bn, _D), lambda i: (i, 0)),
            full((_D, _D)),
            full((1, _D)),
            full((_D, 2 * _D)),
            full((1, 2 * _D)),
            full((1, 2 * _D)),
        ],
        out_specs=pl.BlockSpec((bn, _D), lambda i: (i, 0)),
        out_shape=jax.ShapeDtypeStruct((_N, _D), jnp.float32),
    )(node_feats, cond, film_w, film_b, cond_v, cond_g, cond_b)


def _coeff_body(ef, ev, eg, eb, ew, out):
    v = ev[...]
    norm = jnp.sqrt(jnp.sum(v * v, axis=0, keepdims=True) + 1e-12)
    w = v * (eg[...] / norm)
    p = jnp.tanh(jnp.dot(ef[...], w, preferred_element_type=jnp.float32) + eb[...])
    out[...] = p * ew[...]


def _compute_coeff(ef_pad, edge_v, edge_g, edge_b, ew_pad):
    be = 2048
    grid = (_E_PAD // be,)
    full = lambda shape: pl.BlockSpec(shape, lambda i: (0, 0))
    return pl.pallas_call(
        _coeff_body,
        grid=grid,
        in_specs=[
            pl.BlockSpec((be, 16), lambda i: (i, 0)),
            full((16, _D)),
            full((1, _D)),
            full((1, _D)),
            pl.BlockSpec((be, 1), lambda i: (i, 0)),
        ],
        out_specs=pl.BlockSpec((be, _D), lambda i: (i, 0)),
        out_shape=jax.ShapeDtypeStruct((_E_PAD, _D), jnp.float32),
    )(ef_pad, edge_v, edge_g, edge_b, ew_pad)


def _edge_body(h_hbm, coeff_hbm, comb_hbm, zer_hbm, out_hbm,
               comb_v, src_c, dst_c, hbuf, cbuf, accum,
               gsem0, gsem1, csem0, csem1):
    c = lax.axis_index("c")
    s = lax.axis_index("s")
    w = c * _NS + s
    gsem = (gsem0, gsem1)
    csem = (csem0, csem1)

    pltpu.sync_copy(zer_hbm, accum.at[pl.ds(s * _RPS, _RPS)])
    half = _NCH // 2
    pltpu.sync_copy(comb_hbm.at[pl.ds(w * _NCH, half)], comb_v)
    plsc.subcore_barrier()

    def _decode(j, b):
        r = lax.select(j >= half, j - half, j)
        for cc in range(_CH // 16):
            sl = pl.ds(cc * 16, 16)
            v = comb_v[r, sl]
            src_c[b, sl] = lax.bitwise_and(v, 16383)
            dst_c[b, sl] = lax.shift_right_logical(v, 14)

    def _start(j, b):
        pltpu.async_copy(h_hbm.at[src_c.at[b]], hbuf.at[b], gsem[b])
        pltpu.async_copy(coeff_hbm.at[pl.ds(w * _EPT + j * _CH, _CH)],
                         cbuf.at[b], csem[b])

    for b in range(2):
        _decode(b, b)
        _start(b, b)

    def pair(t, carry):
        for b in range(2):
            j = 2 * t + b
            pltpu.make_async_copy(h_hbm.at[src_c.at[b]], hbuf.at[b],
                                  gsem[b]).wait()
            pltpu.make_async_copy(coeff_hbm.at[pl.ds(w * _EPT + j * _CH, _CH)],
                                  cbuf.at[b], csem[b]).wait()

            def row(i, carry2):
                for cc in range(_D // 16):
                    sl = pl.ds(cc * 16, 16)
                    hbuf[b, i, sl] = hbuf[b, i, sl] * cbuf[b, i, sl]
                return carry2

            lax.fori_loop(0, _CH, row, 0, unroll=2)
            pltpu.sync_copy(hbuf.at[b], accum.at[dst_c.at[b]], add=True)

            jn = j + 2

            @pl.when(jn < _NCH)
            def _():
                @pl.when(jn == half)
                def _reload():
                    pltpu.sync_copy(comb_hbm.at[pl.ds(w * _NCH + half, half)],
                                    comb_v)

                _decode(jn, b)
                _start(jn, b)

        return carry

    lax.fori_loop(0, _NCH // 2, pair, 0, unroll=False)
    plsc.subcore_barrier()
    pltpu.sync_copy(accum.at[pl.ds(s * _RPS, _RPS)],
                    out_hbm.at[c, pl.ds(s * _RPS, _RPS)])


def _edge_scatter(h, coeff, comb2d, zer):
    mesh = plsc.VectorSubcoreMesh(core_axis_name="c", subcore_axis_name="s")
    f = functools.partial(
        pl.kernel,
        out_type=jax.ShapeDtypeStruct((_NC, _N_PAD, _D), jnp.float32),
        mesh=mesh,
        scratch_types=[
            pltpu.VMEM((_NCH // 2, _CH), jnp.int32),
            pltpu.VMEM((2, _CH), jnp.int32),
            pltpu.VMEM((2, _CH), jnp.int32),
            pltpu.VMEM((2, _CH, _D), jnp.float32),
            pltpu.VMEM((2, _CH, _D), jnp.float32),
            pltpu.VMEM_SHARED((_N_PAD, _D), jnp.float32),
            pltpu.SemaphoreType.DMA,
            pltpu.SemaphoreType.DMA,
            pltpu.SemaphoreType.DMA,
            pltpu.SemaphoreType.DMA,
        ],
    )(_edge_body)
    return f(h, coeff, comb2d, zer)


def _sum_body(p0, p1, out):
    out[...] = p0[0] + p1[0]


def _sum_partials(partials):
    bn = 1000
    return pl.pallas_call(
        _sum_body,
        grid=(_N // bn,),
        in_specs=[
            pl.BlockSpec((1, bn, _D), lambda i: (0, i, 0)),
            pl.BlockSpec((1, bn, _D), lambda i: (1, i, 0)),
        ],
        out_specs=pl.BlockSpec((bn, _D), lambda i: (i, 0)),
        out_shape=jax.ShapeDtypeStruct((_N, _D), jnp.float32),
    )(partials, partials)


def kernel(node_feats, cond, edge_feats, edge_index, edge_weights,
           edge_v, edge_g, edge_b, cond_v, cond_g, cond_b, film_w, film_b):
    src = edge_index[0].astype(jnp.int32)
    dst = edge_index[1].astype(jnp.int32)
    pad = _E_PAD - _E
    comb = jnp.bitwise_or(src, jnp.left_shift(dst, 14))
    comb2d = jnp.concatenate([comb, jnp.zeros((pad,), jnp.int32)]).reshape(
        _NW * _NCH, _CH)
    ef_pad = jnp.concatenate(
        [edge_feats, jnp.zeros((pad, edge_feats.shape[1]), jnp.float32)])
    ew_pad = jnp.concatenate([edge_weights, jnp.zeros((pad, 1), jnp.float32)])
    zer = jnp.zeros((_RPS, _D), jnp.float32)

    h = _compute_h(node_feats, cond, film_w, film_b.reshape(1, _D),
                   cond_v, cond_g.reshape(1, 2 * _D), cond_b.reshape(1, 2 * _D))
    coeff = _compute_coeff(ef_pad, edge_v, edge_g.reshape(1, _D),
                           edge_b.reshape(1, _D), ew_pad)
    partials = _edge_scatter(h, coeff, comb2d, zer)
    out = _sum_partials(partials)
    return out


# speedup vs baseline: 1.0516x; 1.0516x over previous
import functools

import jax
import jax.numpy as jnp
from jax import lax
from jax.experimental import pallas as pl
from jax.experimental.pallas import tpu as pltpu
from jax.experimental.pallas import tpu_sc as plsc

_N = 10000
_D = 128
_E = 320000
_NC = 2
_NS = 16
_NW = _NC * _NS
_CH = 64
_NCH = 160
_EPT = _NCH * _CH
_E_PAD = _EPT * _NW
_N_PAD = 10112
_RPS = _N_PAD // _NS


def _h_body(nf, cnd, fw, fb, cv, cg, cb, out):
    hh = jnp.dot(nf[...], fw[...], preferred_element_type=jnp.float32) + fb[...]
    mu = jnp.mean(hh, axis=-1, keepdims=True)
    var = jnp.mean((hh - mu) * (hh - mu), axis=-1, keepdims=True)
    hn = (hh - mu) * lax.rsqrt(var + 1e-5)
    v = cv[...]
    norm = jnp.sqrt(jnp.sum(v * v, axis=0, keepdims=True) + 1e-12)
    w = v * (cg[...] / norm)
    gb = jnp.dot(cnd[...], w, preferred_element_type=jnp.float32) + cb[...]
    gamma = gb[:, :_D] + 1.0
    beta = gb[:, _D:]
    out[...] = jnp.maximum(gamma * hn + beta, 0.0)


def _compute_h(node_feats, cond, film_w, film_b, cond_v, cond_g, cond_b):
    bn = 1000
    grid = (_N // bn,)
    full = lambda shape: pl.BlockSpec(shape, lambda i: (0, 0))
    return pl.pallas_call(
        _h_body,
        grid=grid,
        in_specs=[
            pl.BlockSpec((bn, _D), lambda i: (i, 0)),
            pl.BlockSpec((bn, _D), lambda i: (i, 0)),
            full((_D, _D)),
            full((1, _D)),
            full((_D, 2 * _D)),
            full((1, 2 * _D)),
            full((1, 2 * _D)),
        ],
        out_specs=pl.BlockSpec((bn, _D), lambda i: (i, 0)),
        out_shape=jax.ShapeDtypeStruct((_N, _D), jnp.float32),
    )(node_feats, cond, film_w, film_b, cond_v, cond_g, cond_b)


def _coeff_body(ef, ev, eg, eb, ew, out):
    v = ev[...]
    norm = jnp.sqrt(jnp.sum(v * v, axis=0, keepdims=True) + 1e-12)
    w = v * (eg[...] / norm)
    p = jnp.tanh(jnp.dot(ef[...], w, preferred_element_type=jnp.float32) + eb[...])
    out[...] = p * ew[...]


def _compute_coeff(ef_pad, edge_v, edge_g, edge_b, ew_pad):
    be = 2048
    grid = (_E_PAD // be,)
    full = lambda shape: pl.BlockSpec(shape, lambda i: (0, 0))
    return pl.pallas_call(
        _coeff_body,
        grid=grid,
        in_specs=[
            pl.BlockSpec((be, 16), lambda i: (i, 0)),
            full((16, _D)),
            full((1, _D)),
            full((1, _D)),
            pl.BlockSpec((be, 1), lambda i: (i, 0)),
        ],
        out_specs=pl.BlockSpec((be, _D), lambda i: (i, 0)),
        out_shape=jax.ShapeDtypeStruct((_E_PAD, _D), jnp.float32),
    )(ef_pad, edge_v, edge_g, edge_b, ew_pad)


def _edge_body(h_hbm, coeff_hbm, comb_hbm, zer_hbm, out_hbm,
               comb_v, src_c, dst_c, hbuf, cbuf, accum,
               gsem0, gsem1, csem0, csem1):
    c = lax.axis_index("c")
    s = lax.axis_index("s")
    w = c * _NS + s
    gsem = (gsem0, gsem1)
    csem = (csem0, csem1)

    pltpu.sync_copy(zer_hbm, accum.at[pl.ds(s * _RPS, _RPS)])
    half = _NCH // 2
    pltpu.sync_copy(comb_hbm.at[pl.ds(w * _NCH, half)], comb_v)
    plsc.subcore_barrier()

    def _decode(j, b):
        r = lax.select(j >= half, j - half, j)
        for cc in range(_CH // 16):
            sl = pl.ds(cc * 16, 16)
            v = comb_v[r, sl]
            src_c[b, sl] = lax.bitwise_and(v, 16383)
            dst_c[b, sl] = lax.shift_right_logical(v, 14)

    def _start(j, b):
        pltpu.async_copy(h_hbm.at[src_c.at[b]], hbuf.at[b], gsem[b])
        pltpu.async_copy(coeff_hbm.at[pl.ds(w * _EPT + j * _CH, _CH)],
                         cbuf.at[b], csem[b])

    for b in range(2):
        _decode(b, b)
        _start(b, b)

    def pair(t, carry):
        for b in range(2):
            j = 2 * t + b
            pltpu.make_async_copy(h_hbm.at[src_c.at[b]], hbuf.at[b],
                                  gsem[b]).wait()
            pltpu.make_async_copy(coeff_hbm.at[pl.ds(w * _EPT + j * _CH, _CH)],
                                  cbuf.at[b], csem[b]).wait()

            def row(i, carry2):
                for cc in range(_D // 16):
                    sl = pl.ds(cc * 16, 16)
                    hbuf[b, i, sl] = hbuf[b, i, sl] * cbuf[b, i, sl]
                return carry2

            lax.fori_loop(0, _CH, row, 0, unroll=False)
            pltpu.sync_copy(hbuf.at[b], accum.at[dst_c.at[b]], add=True)

            jn = j + 2

            @pl.when(jn < _NCH)
            def _():
                @pl.when(jn == half)
                def _reload():
                    pltpu.sync_copy(comb_hbm.at[pl.ds(w * _NCH + half, half)],
                                    comb_v)

                _decode(jn, b)
                _start(jn, b)

        return carry

    lax.fori_loop(0, _NCH // 2, pair, 0, unroll=False)
    plsc.subcore_barrier()
    pltpu.sync_copy(accum.at[pl.ds(s * _RPS, _RPS)],
                    out_hbm.at[c, pl.ds(s * _RPS, _RPS)])


def _edge_scatter(h, coeff, comb2d, zer):
    mesh = plsc.VectorSubcoreMesh(core_axis_name="c", subcore_axis_name="s")
    f = functools.partial(
        pl.kernel,
        out_type=jax.ShapeDtypeStruct((_NC, _N_PAD, _D), jnp.float32),
        mesh=mesh,
        scratch_types=[
            pltpu.VMEM((_NCH // 2, _CH), jnp.int32),
            pltpu.VMEM((2, _CH), jnp.int32),
            pltpu.VMEM((2, _CH), jnp.int32),
            pltpu.VMEM((2, _CH, _D), jnp.float32),
            pltpu.VMEM((2, _CH, _D), jnp.float32),
            pltpu.VMEM_SHARED((_N_PAD, _D), jnp.float32),
            pltpu.SemaphoreType.DMA,
            pltpu.SemaphoreType.DMA,
            pltpu.SemaphoreType.DMA,
            pltpu.SemaphoreType.DMA,
        ],
    )(_edge_body)
    return f(h, coeff, comb2d, zer)


def _sum_body(p0, p1, out):
    out[...] = p0[0] + p1[0]


def _sum_partials(partials):
    bn = 1000
    return pl.pallas_call(
        _sum_body,
        grid=(_N // bn,),
        in_specs=[
            pl.BlockSpec((1, bn, _D), lambda i: (0, i, 0)),
            pl.BlockSpec((1, bn, _D), lambda i: (1, i, 0)),
        ],
        out_specs=pl.BlockSpec((bn, _D), lambda i: (i, 0)),
        out_shape=jax.ShapeDtypeStruct((_N, _D), jnp.float32),
    )(partials, partials)


def kernel(node_feats, cond, edge_feats, edge_index, edge_weights,
           edge_v, edge_g, edge_b, cond_v, cond_g, cond_b, film_w, film_b):
    src = edge_index[0].astype(jnp.int32)
    dst = edge_index[1].astype(jnp.int32)
    pad = _E_PAD - _E
    comb = jnp.bitwise_or(src, jnp.left_shift(dst, 14))
    comb2d = jnp.concatenate([comb, jnp.zeros((pad,), jnp.int32)]).reshape(
        _NW * _NCH, _CH)
    ef_pad = jnp.concatenate(
        [edge_feats, jnp.zeros((pad, edge_feats.shape[1]), jnp.float32)])
    ew_pad = jnp.concatenate([edge_weights, jnp.zeros((pad, 1), jnp.float32)])
    zer = jnp.zeros((_RPS, _D), jnp.float32)

    h = _compute_h(node_feats, cond, film_w, film_b.reshape(1, _D),
                   cond_v, cond_g.reshape(1, 2 * _D), cond_b.reshape(1, 2 * _D))
    coeff = _compute_coeff(ef_pad, edge_v, edge_g.reshape(1, _D),
                           edge_b.reshape(1, _D), ew_pad)
    partials = _edge_scatter(h, coeff, comb2d, zer)
    out = _sum_partials(partials)
    return out
